# Initial kernel scaffold; baseline (speedup 1.0000x reference)
import jax
import jax.numpy as jnp
from jax.experimental import pallas as pl


def kernel(x, W1, W2, W3, W4, W5, L1, L2w, L2b, L3w, L3b, L4w, L4b, L5w, L5b):
    raise NotImplementedError("write your pallas kernel here")


# speedup vs baseline: 6.1811x; 6.1811x over previous
import functools

import jax
import jax.numpy as jnp
from jax import lax
from jax.experimental import pallas as pl
from jax.experimental.pallas import tpu as pltpu
from jax.experimental.pallas import tpu_sc as plsc

K = 20
EPS = 1e-5
NEG = -3.0e38


def _lrelu(x):
    return jnp.where(x >= 0, x, 0.2 * x)


def _knn_feat(XT, W, blk=128):
    B, N, C = XT.shape
    O = W.shape[0]

    def body(xr_ref, xb_ref, w_ref, idx_ref, g_ref, c_ref):
        b = pl.program_id(0)
        xr = xr_ref[0]
        xb = xb_ref[0]
        w = w_ref[...]
        dn = (((1,), (1,)), ((), ()))
        g_ref[0] = lax.dot_general(xr, w[:, :C], dn,
                                   preferred_element_type=jnp.float32)
        c_ref[0] = lax.dot_general(xr, w[:, C:], dn,
                                   preferred_element_type=jnp.float32)
        in2 = lax.dot_general(xr, xb, (((1,), (0,)), ((), ())),
                              preferred_element_type=jnp.float32)
        xx_row = jnp.sum(xb * xb, axis=0, keepdims=True)
        xx_col = jnp.sum(xr * xr, axis=1, keepdims=True)
        pd = (2.0 * in2 - xx_row) - xx_col
        iota = lax.broadcasted_iota(jnp.int32, (blk, N), 1)
        cols = []
        for _ in range(K):
            m = jnp.max(pd, axis=1, keepdims=True)
            hit = pd == m
            idxk = jnp.min(jnp.where(hit, iota, N), axis=1, keepdims=True)
            cols.append(idxk)
            pd = jnp.where(iota == idxk, NEG, pd)
        idx = jnp.concatenate(cols, axis=1) + b * N
        idx_ref[0] = idx

    return pl.pallas_call(
        body,
        grid=(B, N // blk),
        in_specs=[
            pl.BlockSpec((1, blk, C), lambda b, i: (b, i, 0)),
            pl.BlockSpec((1, C, N), lambda b, i: (b, 0, 0)),
            pl.BlockSpec((O, 2 * C), lambda b, i: (0, 0)),
        ],
        out_specs=[
            pl.BlockSpec((1, blk, K), lambda b, i: (b, i, 0)),
            pl.BlockSpec((1, blk, O), lambda b, i: (b, i, 0)),
            pl.BlockSpec((1, blk, O), lambda b, i: (b, i, 0)),
        ],
        out_shape=[
            jax.ShapeDtypeStruct((B, N, K), jnp.int32),
            jax.ShapeDtypeStruct((B, N, O), jnp.float32),
            jax.ShapeDtypeStruct((B, N, O), jnp.float32),
        ],
    )(XT, jnp.transpose(XT, (0, 2, 1)), W)


def _gather_reduce(gT, idxf, O):
    R = gT.shape[0]
    info = plsc.get_sparse_core_info()
    NC, NS = info.num_cores, info.num_subcores
    NW = NC * NS
    PW = R // NW
    P = 4
    CH = PW // P

    mesh = plsc.VectorSubcoreMesh(core_axis_name="c", subcore_axis_name="s")

    @functools.partial(
        pl.kernel,
        out_type=[
            jax.ShapeDtypeStruct((R, O), jnp.float32),
            jax.ShapeDtypeStruct((R, O), jnp.float32),
            jax.ShapeDtypeStruct((NW, O), jnp.float32),
        ],
        mesh=mesh,
        compiler_params=pltpu.CompilerParams(use_tc_tiling_on_sc=False),
        scratch_types=[
            pltpu.VMEM((P * K,), jnp.int32),
            pltpu.VMEM((P * K, O), jnp.float32),
            pltpu.VMEM((P, O), jnp.float32),
            pltpu.VMEM((P, O), jnp.float32),
            pltpu.VMEM((O,), jnp.float32),
            pltpu.SemaphoreType.DMA,
        ],
    )
    def run(g_hbm, idx_hbm, pmax_hbm, tsum_hbm, qpart_hbm,
            idx_v, rows_v, pm_v, ts_v, q_v, sem):
        wid = lax.axis_index("s") * NC + lax.axis_index("c")
        zeros16 = jnp.zeros((16,), jnp.float32)

        def zq(j, carry):
            q_v[pl.ds(j * 16, 16)] = zeros16
            return carry

        lax.fori_loop(0, O // 16, zq, 0)

        def chunk(ch, carry):
            base = wid * PW + ch * P
            pltpu.sync_copy(idx_hbm.at[pl.ds(base * K, P * K)], idx_v)
            pltpu.async_copy(g_hbm.at[idx_v], rows_v, sem).wait()

            def colj(j, c2):
                o0 = j * 16
                for p in range(P):
                    r0 = p * K
                    v = rows_v[r0, pl.ds(o0, 16)]
                    mx = v
                    sm = v
                    sq = v * v
                    for kk in range(1, K):
                        v = rows_v[r0 + kk, pl.ds(o0, 16)]
                        mx = jnp.maximum(mx, v)
                        sm = sm + v
                        sq = sq + v * v
                    pm_v[p, pl.ds(o0, 16)] = mx
                    ts_v[p, pl.ds(o0, 16)] = sm
                    q_v[pl.ds(o0, 16)] = q_v[pl.ds(o0, 16)] + sq
                return c2

            lax.fori_loop(0, O // 16, colj, 0)
            pltpu.sync_copy(pm_v, pmax_hbm.at[pl.ds(base, P)])
            pltpu.sync_copy(ts_v, tsum_hbm.at[pl.ds(base, P)])
            return carry

        lax.fori_loop(0, CH, chunk, 0)
        pltpu.sync_copy(q_v, qpart_hbm.at[wid])

    return run(gT, idxf)


def _sc_gather(table, idxf):
    R, C = table.shape
    T = idxf.shape[0]
    info = plsc.get_sparse_core_info()
    NC, NS = info.num_cores, info.num_subcores
    NW = NC * NS
    PW = (T // K) // NW
    P = 4
    CH = PW // P

    mesh = plsc.VectorSubcoreMesh(core_axis_name="c", subcore_axis_name="s")

    @functools.partial(
        pl.kernel,
        out_type=jax.ShapeDtypeStruct((T, C), jnp.float32),
        mesh=mesh,
        compiler_params=pltpu.CompilerParams(use_tc_tiling_on_sc=False),
        scratch_types=[
            pltpu.VMEM((P * K,), jnp.int32),
            pltpu.VMEM((P * K, C), jnp.float32),
            pltpu.SemaphoreType.DMA,
        ],
    )
    def run(t_hbm, idx_hbm, out_hbm, idx_v, rows_v, sem):
        wid = lax.axis_index("s") * NC + lax.axis_index("c")

        def chunk(ch, carry):
            base = (wid * PW + ch * P) * K
            pltpu.sync_copy(idx_hbm.at[pl.ds(base, P * K)], idx_v)
            pltpu.async_copy(t_hbm.at[idx_v], rows_v, sem).wait()
            pltpu.sync_copy(rows_v, out_hbm.at[pl.ds(base, P * K)])
            return carry

        lax.fori_loop(0, CH, chunk, 0)

    return run(table, idxf)


def _econv_exact(fnb, xf, W, Cf, pts=64):
    T, Cp = fnb.shape
    R = xf.shape[0]
    O = W.shape[0]
    nb = R // pts

    def body(f_ref, x_ref, w_ref, pm_ref, s1_ref, s2_ref):
        fn = f_ref[...][:, :Cf]
        xc = x_ref[...][:, :Cf]
        xr = jnp.broadcast_to(
            xc[:, None, :], (pts, K, Cf)).reshape(pts * K, Cf)
        f = jnp.concatenate([fn, xr], axis=1)
        h = lax.dot_general(f, w_ref[...], (((1,), (1,)), ((), ())),
                            preferred_element_type=jnp.float32)
        h3 = h.reshape(pts, K, O)
        s1_ref[0] = jnp.sum(jnp.sum(h3, axis=1), axis=0, keepdims=True)
        s2_ref[0] = jnp.sum(jnp.sum(h3 * h3, axis=1), axis=0, keepdims=True)
        pm_ref[...] = jnp.max(h3, axis=1)

    return pl.pallas_call(
        body,
        grid=(nb,),
        in_specs=[
            pl.BlockSpec((pts * K, Cp), lambda i: (i, 0)),
            pl.BlockSpec((pts, Cp), lambda i: (i, 0)),
            pl.BlockSpec((O, 2 * Cf), lambda i: (0, 0)),
        ],
        out_specs=[
            pl.BlockSpec((pts, O), lambda i: (i, 0)),
            pl.BlockSpec((1, 1, O), lambda i: (i, 0, 0)),
            pl.BlockSpec((1, 1, O), lambda i: (i, 0, 0)),
        ],
        out_shape=[
            jax.ShapeDtypeStruct((R, O), jnp.float32),
            jax.ShapeDtypeStruct((nb, 1, O), jnp.float32),
            jax.ShapeDtypeStruct((nb, 1, O), jnp.float32),
        ],
    )(fnb, xf, W)


def _econv_var(fnb, xf, W, Cf, s1p, M, pts=64):
    T, Cp = fnb.shape
    R = xf.shape[0]
    O = W.shape[0]
    nb = R // pts
    nbp = s1p.shape[0]

    def body(f_ref, x_ref, w_ref, s1_ref, sc_ref):
        mean = jnp.sum(s1_ref[:, 0, :], axis=0, keepdims=True) / M
        fn = f_ref[...][:, :Cf]
        xc = x_ref[...][:, :Cf]
        xr = jnp.broadcast_to(
            xc[:, None, :], (pts, K, Cf)).reshape(pts * K, Cf)
        f = jnp.concatenate([fn, xr], axis=1)
        h = lax.dot_general(f, w_ref[...], (((1,), (1,)), ((), ())),
                            preferred_element_type=jnp.float32)
        d = h - mean
        d3 = (d * d).reshape(pts, K, O)
        sc_ref[0] = jnp.sum(jnp.sum(d3, axis=1), axis=0, keepdims=True)

    return pl.pallas_call(
        body,
        grid=(nb,),
        in_specs=[
            pl.BlockSpec((pts * K, Cp), lambda i: (i, 0)),
            pl.BlockSpec((pts, Cp), lambda i: (i, 0)),
            pl.BlockSpec((O, 2 * Cf), lambda i: (0, 0)),
            pl.BlockSpec((nbp, 1, O), lambda i: (0, 0, 0)),
        ],
        out_specs=pl.BlockSpec((1, 1, O), lambda i: (i, 0, 0)),
        out_shape=jax.ShapeDtypeStruct((nb, 1, O), jnp.float32),
    )(fnb, xf, W, s1p)


def _normalize2(pmax, s1p, s2p, M, blk=1024):
    R, O = pmax.shape
    nb = s1p.shape[0]

    def body(p_ref, s1_ref, s2_ref, o_ref):
        mean = jnp.sum(s1_ref[:, 0, :], axis=0, keepdims=True) / M
        var = jnp.sum(s2_ref[:, 0, :], axis=0, keepdims=True) / M
        y = (p_ref[...] - mean) / jnp.sqrt(var + EPS)
        o_ref[...] = _lrelu(y)

    return pl.pallas_call(
        body,
        grid=(R // blk,),
        in_specs=[
            pl.BlockSpec((blk, O), lambda i: (i, 0)),
            pl.BlockSpec((nb, 1, O), lambda i: (0, 0, 0)),
            pl.BlockSpec((nb, 1, O), lambda i: (0, 0, 0)),
        ],
        out_specs=pl.BlockSpec((blk, O), lambda i: (i, 0)),
        out_shape=jax.ShapeDtypeStruct((R, O), jnp.float32),
    )(pmax, s1p, s2p)


def _edgeconv_exact(XT, W, Cf):
    B, N, C = XT.shape
    O = W.shape[0]
    idx, _, _ = _knn_feat(XT, W)
    if C < 8:
        table = jnp.pad(XT.reshape(B * N, C), ((0, 0), (0, 8 - C)))
    else:
        table = XT.reshape(B * N, C)
    fnb = _sc_gather(table, idx.reshape(B * N * K))
    M = float(B * N * K)
    pmax, s1, _ = _econv_exact(fnb, table, W, Cf)
    s2c = _econv_var(fnb, table, W, Cf, s1, M)
    xn = _normalize2(pmax, s1, s2c, M)
    return xn.reshape(B, N, O)


def _stats(tsum, cT, blk=1024):
    R, O = tsum.shape

    def body(t_ref, c_ref, s1_ref, s2_ref):
        @pl.when(pl.program_id(0) == 0)
        def _():
            s1_ref[...] = jnp.zeros_like(s1_ref)
            s2_ref[...] = jnp.zeros_like(s2_ref)

        t = t_ref[...]
        c = c_ref[...]
        s1_ref[...] += jnp.sum(t + K * c, axis=0, keepdims=True)
        s2_ref[...] += jnp.sum(2.0 * c * t + K * c * c, axis=0, keepdims=True)

    return pl.pallas_call(
        body,
        grid=(R // blk,),
        in_specs=[
            pl.BlockSpec((blk, O), lambda i: (i, 0)),
            pl.BlockSpec((blk, O), lambda i: (i, 0)),
        ],
        out_specs=[
            pl.BlockSpec((1, O), lambda i: (0, 0)),
            pl.BlockSpec((1, O), lambda i: (0, 0)),
        ],
        out_shape=[
            jax.ShapeDtypeStruct((1, O), jnp.float32),
            jax.ShapeDtypeStruct((1, O), jnp.float32),
        ],
    )(tsum, cT)


def _normalize(pmax, cT, s1, s2, qpart, M, blk=1024):
    R, O = pmax.shape
    NW = qpart.shape[0]

    def body(p_ref, c_ref, s1_ref, s2_ref, q_ref, o_ref):
        q = jnp.sum(q_ref[...], axis=0, keepdims=True)
        mean = s1_ref[...] / M
        ex2 = (s2_ref[...] + q) / M
        var = ex2 - mean * mean
        y = (p_ref[...] + c_ref[...] - mean) / jnp.sqrt(var + EPS)
        o_ref[...] = _lrelu(y)

    return pl.pallas_call(
        body,
        grid=(R // blk,),
        in_specs=[
            pl.BlockSpec((blk, O), lambda i: (i, 0)),
            pl.BlockSpec((blk, O), lambda i: (i, 0)),
            pl.BlockSpec((1, O), lambda i: (0, 0)),
            pl.BlockSpec((1, O), lambda i: (0, 0)),
            pl.BlockSpec((NW, O), lambda i: (0, 0)),
        ],
        out_specs=pl.BlockSpec((blk, O), lambda i: (i, 0)),
        out_shape=jax.ShapeDtypeStruct((R, O), jnp.float32),
    )(pmax, cT, s1, s2, qpart)


def _edgeconv(XT, W):
    B, N, C = XT.shape
    O = W.shape[0]
    idx, gT, cT = _knn_feat(XT, W)
    pmax, tsum, qpart = _gather_reduce(
        gT.reshape(B * N, O), idx.reshape(B * N * K), O)
    cf = cT.reshape(B * N, O)
    s1, s2 = _stats(tsum, cf)
    xn = _normalize(pmax, cf, s1, s2, qpart, float(B * N * K))
    return xn.reshape(B, N, O)


def _l5a(xs, W5, B, N, blk=512):
    R = B * N
    Cs = [x.shape[1] for x in xs]
    O5 = W5.shape[0]
    nb = N // blk
    offs = [0]
    for c in Cs:
        offs.append(offs[-1] + c)

    def body(x1_ref, x2_ref, x3_ref, x4_ref, w_ref, h_ref, s1_ref, s2_ref,
             hm_ref):
        b = pl.program_id(0)
        j = pl.program_id(1)

        @pl.when((b == 0) & (j == 0))
        def _():
            s1_ref[...] = jnp.zeros_like(s1_ref)
            s2_ref[...] = jnp.zeros_like(s2_ref)

        @pl.when(j == 0)
        def _():
            hm_ref[0] = jnp.full((1, O5), NEG, jnp.float32)

        w = w_ref[...]
        dn = (((1,), (1,)), ((), ()))
        h = lax.dot_general(x1_ref[...], w[:, offs[0]:offs[1]], dn,
                            preferred_element_type=jnp.float32)
        h += lax.dot_general(x2_ref[...], w[:, offs[1]:offs[2]], dn,
                             preferred_element_type=jnp.float32)
        h += lax.dot_general(x3_ref[...], w[:, offs[2]:offs[3]], dn,
                             preferred_element_type=jnp.float32)
        h += lax.dot_general(x4_ref[...], w[:, offs[3]:offs[4]], dn,
                             preferred_element_type=jnp.float32)
        h_ref[...] = h
        s1_ref[...] += jnp.sum(h, axis=0, keepdims=True)
        s2_ref[...] += jnp.sum(h * h, axis=0, keepdims=True)
        hm_ref[0] = jnp.maximum(hm_ref[0], jnp.max(h, 0, keepdims=True))

    rowmap = lambda b, j: (b * nb + j, 0)
    return pl.pallas_call(
        body,
        grid=(B, nb),
        in_specs=[pl.BlockSpec((blk, c), rowmap) for c in Cs]
        + [pl.BlockSpec((O5, offs[-1]), lambda b, j: (0, 0))],
        out_specs=[
            pl.BlockSpec((blk, O5), rowmap),
            pl.BlockSpec((1, O5), lambda b, j: (0, 0)),
            pl.BlockSpec((1, O5), lambda b, j: (0, 0)),
            pl.BlockSpec((1, 1, O5), lambda b, j: (b, 0, 0)),
        ],
        out_shape=[
            jax.ShapeDtypeStruct((R, O5), jnp.float32),
            jax.ShapeDtypeStruct((1, O5), jnp.float32),
            jax.ShapeDtypeStruct((1, O5), jnp.float32),
            jax.ShapeDtypeStruct((B, 1, O5), jnp.float32),
        ],
    )(*xs, W5)


def _l5b(h5, s1, s2, B, N, blk=512):
    R, O5 = h5.shape
    nb = N // blk
    M5 = float(R)

    def body(h_ref, s1_ref, s2_ref, p2_ref):
        j = pl.program_id(1)

        @pl.when(j == 0)
        def _():
            p2_ref[0] = jnp.zeros((1, O5), jnp.float32)

        mean = s1_ref[...] / M5
        var = s2_ref[...] / M5 - mean * mean
        y = _lrelu((h_ref[...] - mean) / jnp.sqrt(var + EPS))
        p2_ref[0] += jnp.sum(y, axis=0, keepdims=True) / N

    return pl.pallas_call(
        body,
        grid=(B, nb),
        in_specs=[
            pl.BlockSpec((blk, O5), lambda b, j: (b * nb + j, 0)),
            pl.BlockSpec((1, O5), lambda b, j: (0, 0)),
            pl.BlockSpec((1, O5), lambda b, j: (0, 0)),
        ],
        out_specs=pl.BlockSpec((1, 1, O5), lambda b, j: (b, 0, 0)),
        out_shape=jax.ShapeDtypeStruct((B, 1, O5), jnp.float32),
    )(h5, s1, s2)


def _head(hmax, p2, s1, s2, M5, L1, L2w, L2b, L3w, L3b, L4w, L4b, L5w, L5b):
    B = hmax.shape[0]

    def bn0(z):
        m = jnp.mean(z, axis=0, keepdims=True)
        v = jnp.mean((z - m) * (z - m), axis=0, keepdims=True)
        return (z - m) / jnp.sqrt(v + EPS)

    def body(hm_ref, p2_ref, s1_ref, s2_ref, l1_ref, l2w_ref, l2b_ref,
             l3w_ref, l3b_ref, l4w_ref, l4b_ref, l5w_ref, l5b_ref, o_ref):
        mean = s1_ref[...] / M5
        var = s2_ref[...] / M5 - mean * mean
        p1 = _lrelu((hm_ref[...] - mean) / jnp.sqrt(var + EPS))
        h = jnp.concatenate([p1, p2_ref[...]], axis=1)
        dn = (((1,), (1,)), ((), ()))
        z = lax.dot_general(h, l1_ref[...], dn,
                            preferred_element_type=jnp.float32)
        h = _lrelu(bn0(z))
        z = lax.dot_general(h, l2w_ref[...], dn,
                            preferred_element_type=jnp.float32) + l2b_ref[...]
        h = _lrelu(bn0(z))
        h = lax.dot_general(h, l3w_ref[...], dn,
                            preferred_element_type=jnp.float32) + l3b_ref[...]
        h = lax.dot_general(h, l4w_ref[...], dn,
                            preferred_element_type=jnp.float32) + l4b_ref[...]
        h = lax.dot_general(h, l5w_ref[...], dn,
                            preferred_element_type=jnp.float32) + l5b_ref[...]
        o_ref[...] = h

    args = [hmax, p2, s1, s2, L1, L2w, L2b.reshape(1, -1), L3w,
            L3b.reshape(1, -1), L4w, L4b.reshape(1, -1), L5w,
            L5b.reshape(1, -1)]
    return pl.pallas_call(
        body,
        in_specs=[pl.BlockSpec(a.shape, lambda: (0,) * a.ndim) for a in args],
        out_specs=pl.BlockSpec((B, 40), lambda: (0, 0)),
        out_shape=jax.ShapeDtypeStruct((B, 40), jnp.float32),
    )(*args)


def kernel(x, W1, W2, W3, W4, W5, L1, L2w, L2b, L3w, L3b, L4w, L4b, L5w,
           L5b):
    B, _, N = x.shape
    XT = jnp.transpose(x, (0, 2, 1))
    x1 = _edgeconv_exact(XT, W1, 3)
    x2 = _edgeconv_exact(x1, W2, 64)
    x3 = _edgeconv_exact(x2, W3, 64)
    x4 = _edgeconv(x3, W4)
    xs = [x1.reshape(B * N, -1), x2.reshape(B * N, -1),
          x3.reshape(B * N, -1), x4.reshape(B * N, -1)]
    h5, s1, s2, hmax = _l5a(xs, W5, B, N)
    p2 = _l5b(h5, s1, s2, B, N)
    return _head(hmax.reshape(B, -1), p2.reshape(B, -1), s1, s2,
                 float(B * N), L1, L2w, L2b, L3w, L3b, L4w, L4b, L5w, L5b)
